# baseline (speedup 1.0000x reference)
import functools

import jax
import jax.numpy as jnp
from jax import lax
from jax.experimental import pallas as pl
from jax.experimental.pallas import tpu as pltpu
from jax.experimental.pallas import tpu_sc as plsc

_dot = functools.partial(jnp.dot, precision=jax.lax.Precision.HIGHEST)

N_NODES = 10000
N_EDGES = 320000
E2 = N_EDGES // 2
LATENT = 64

NC = 2
NS = 16
NW = NC * NS
E_PER_W = N_EDGES // NW
G_CH = 200
G_NCH = E_PER_W // G_CH
S_CH = 200
S_NCH = E_PER_W // S_CH
N_ACC = 10240
N_PER_S = N_ACC // NS

EB2 = 3200
NB = 2000

_SC_PARAMS = pltpu.CompilerParams(use_tc_tiling_on_sc=False)


def _mesh():
    return plsc.VectorSubcoreMesh(core_axis_name="c", subcore_axis_name="s",
                                  num_cores=NC, num_subcores=NS)


@functools.partial(
    pl.kernel,
    out_type=(
        jax.ShapeDtypeStruct((N_EDGES, LATENT), jnp.float32),
        jax.ShapeDtypeStruct((N_EDGES, LATENT), jnp.float32),
    ),
    mesh=_mesh(),
    scratch_types=[
        pltpu.VMEM((G_NCH, G_CH), jnp.int32),
        pltpu.VMEM((G_NCH, G_CH), jnp.int32),
        pltpu.VMEM((2, G_CH, LATENT), jnp.float32),
        pltpu.VMEM((2, G_CH, LATENT), jnp.float32),
        pltpu.SemaphoreType.DMA,
        pltpu.SemaphoreType.DMA,
        pltpu.SemaphoreType.DMA,
        pltpu.SemaphoreType.DMA,
        pltpu.SemaphoreType.DMA,
        pltpu.SemaphoreType.DMA,
        pltpu.SemaphoreType.DMA,
        pltpu.SemaphoreType.DMA,
    ],
    compiler_params=_SC_PARAMS,
)
def _sc_gather(tab, snd_h, rcv_h, oa, ob, ia, ib, ba, bb,
               sa0, sb0, sa1, sb1, wa0, wb0, wa1, wb1):
    wid = lax.axis_index("s") * NC + lax.axis_index("c")
    pltpu.sync_copy(snd_h.at[wid], ia)
    pltpu.sync_copy(rcv_h.at[wid], ib)
    base = wid * E_PER_W

    def body(g, carry):
        i0 = 2 * g
        i1 = i0 + 1
        off0 = base + i0 * G_CH
        off1 = base + i1 * G_CH
        ca0 = pltpu.async_copy(tab.at[ia.at[i0]], ba.at[0], sa0)
        cb0 = pltpu.async_copy(tab.at[ib.at[i0]], bb.at[0], sb0)
        ca1 = pltpu.async_copy(tab.at[ia.at[i1]], ba.at[1], sa1)
        cb1 = pltpu.async_copy(tab.at[ib.at[i1]], bb.at[1], sb1)
        ca0.wait()
        cb0.wait()
        va0 = pltpu.async_copy(ba.at[0], oa.at[pl.ds(off0, G_CH)], wa0)
        vb0 = pltpu.async_copy(bb.at[0], ob.at[pl.ds(off0, G_CH)], wb0)
        ca1.wait()
        cb1.wait()
        va1 = pltpu.async_copy(ba.at[1], oa.at[pl.ds(off1, G_CH)], wa1)
        vb1 = pltpu.async_copy(bb.at[1], ob.at[pl.ds(off1, G_CH)], wb1)
        va0.wait()
        vb0.wait()
        va1.wait()
        vb1.wait()
        return carry

    lax.fori_loop(0, G_NCH // 2, body, 0)


@functools.partial(
    pl.kernel,
    out_type=(
        jax.ShapeDtypeStruct((N_ACC, LATENT), jnp.float32),
        jax.ShapeDtypeStruct((N_ACC, LATENT), jnp.float32),
    ),
    mesh=_mesh(),
    scratch_types=[
        pltpu.VMEM((S_NCH, S_CH), jnp.int32),
        pltpu.VMEM((2, S_CH, LATENT), jnp.float32),
        pltpu.VMEM_SHARED((N_ACC, LATENT), jnp.float32),
        pltpu.SemaphoreType.DMA,
        pltpu.SemaphoreType.DMA,
        pltpu.SemaphoreType.DMA,
        pltpu.SemaphoreType.DMA,
    ],
    compiler_params=_SC_PARAMS,
)
def _sc_segsum(vals_h, rcv_h, zeros_h, o0, o1, idx, buf, acc,
               l0, l1, t0, t1):
    cid = lax.axis_index("c")
    sid = lax.axis_index("s")
    wid = sid * NC + cid
    rows = pl.ds(sid * N_PER_S, N_PER_S)
    pltpu.sync_copy(zeros_h.at[rows], acc.at[rows])
    pltpu.sync_copy(rcv_h.at[wid], idx)
    plsc.subcore_barrier()
    base = wid * E_PER_W

    def body(g, carry):
        i0 = 2 * g
        i1 = i0 + 1
        cl0 = pltpu.async_copy(vals_h.at[pl.ds(base + i0 * S_CH, S_CH)],
                               buf.at[0], l0)
        cl1 = pltpu.async_copy(vals_h.at[pl.ds(base + i1 * S_CH, S_CH)],
                               buf.at[1], l1)
        cl0.wait()
        cs0 = pltpu.async_copy(buf.at[0], acc.at[idx.at[i0]], t0, add=True)
        cl1.wait()
        cs1 = pltpu.async_copy(buf.at[1], acc.at[idx.at[i1]], t1, add=True)
        cs0.wait()
        cs1.wait()
        return carry

    lax.fori_loop(0, S_NCH // 2, body, 0)
    plsc.subcore_barrier()

    @pl.when(cid == 0)
    def _():
        pltpu.sync_copy(acc.at[rows], o0.at[rows])

    @pl.when(cid == 1)
    def _():
        pltpu.sync_copy(acc.at[rows], o1.at[rows])


def _tc_embed_node(nodes, w1, b1, w2, b2, ws, wr):
    def body(x, w1r, b1r, w2r, b2r, wsr, wrr, hn, ab):
        h = jax.nn.gelu(_dot(x[...], w1r[...]) + b1r[...])
        hv = _dot(h, w2r[...]) + b2r[...]
        hn[...] = hv
        ab[...] = jnp.concatenate(
            [_dot(hv, wsr[...]), _dot(hv, wrr[...])], axis=-1)

    c = lambda i: (0, 0)
    r = lambda i: (i, 0)
    return pl.pallas_call(
        body,
        grid=(N_NODES // NB,),
        in_specs=[
            pl.BlockSpec((NB, 128), r),
            pl.BlockSpec((128, 64), c), pl.BlockSpec((1, 64), c),
            pl.BlockSpec((64, 64), c), pl.BlockSpec((1, 64), c),
            pl.BlockSpec((64, 64), c), pl.BlockSpec((64, 64), c),
        ],
        out_specs=[pl.BlockSpec((NB, 64), r), pl.BlockSpec((NB, 128), r)],
        out_shape=[jax.ShapeDtypeStruct((N_NODES, 64), jnp.float32),
                   jax.ShapeDtypeStruct((N_NODES, 128), jnp.float32)],
    )(nodes, w1, b1, w2, b2, ws, wr)


def _tc_embed_edge(edges8t, w1p, b1p, w2p, b2p):
    def body(x, w1r, b1r, w2r, b2r, he):
        pre = lax.dot_general(x[...], w1r[...], (((0,), (0,)), ((), ())),
                              precision=jax.lax.Precision.HIGHEST)
        h = jax.nn.gelu(pre + b1r[...])
        he[...] = _dot(h, w2r[...]) + b2r[...]

    c = lambda i: (0, 0)
    r = lambda i: (i, 0)
    return pl.pallas_call(
        body,
        grid=(E2 // EB2,),
        in_specs=[
            pl.BlockSpec((8, EB2), lambda i: (0, i)),
            pl.BlockSpec((8, 128), c), pl.BlockSpec((1, 128), c),
            pl.BlockSpec((128, 128), c), pl.BlockSpec((1, 128), c),
        ],
        out_specs=pl.BlockSpec((EB2, 128), r),
        out_shape=jax.ShapeDtypeStruct((E2, 128), jnp.float32),
    )(edges8t, w1p, b1p, w2p, b2p)


def _tc_edge(he_p, ga_p, gb_p, w1p, c0p, w2p, b2p):
    def body(he, gar, gbr, w1r, c0r, w2r, b2r, out):
        hev = he[...]
        pre = _dot(hev, w1r[...]) + (gar[...] + gbr[...]) + c0r[...]
        t = jax.nn.gelu(pre)
        out[...] = _dot(t, w2r[...]) + b2r[...] + hev

    c = lambda i: (0, 0)
    r = lambda i: (i, 0)
    return pl.pallas_call(
        body,
        grid=(E2 // EB2,),
        in_specs=[
            pl.BlockSpec((EB2, 128), r), pl.BlockSpec((EB2, 128), r),
            pl.BlockSpec((EB2, 128), r),
            pl.BlockSpec((128, 128), c), pl.BlockSpec((1, 128), c),
            pl.BlockSpec((128, 128), c), pl.BlockSpec((1, 128), c),
        ],
        out_specs=pl.BlockSpec((EB2, 128), r),
        out_shape=jax.ShapeDtypeStruct((E2, 128), jnp.float32),
    )(he_p, ga_p, gb_p, w1p, c0p, w2p, b2p)


def _node_core(hnv, r0, r1, v1n, v1r, c1, v2, d2, gam, bet):
    rec = r0[...] + r1[...]
    t = jax.nn.gelu(_dot(hnv, v1n[...]) + _dot(rec, v1r[...]) + c1[...])
    y = _dot(t, v2[...]) + d2[...] + hnv
    m = jnp.mean(y, axis=-1, keepdims=True)
    v = jnp.mean((y - m) ** 2, axis=-1, keepdims=True)
    return (y - m) / jnp.sqrt(v + 1e-6) * gam[...] + bet[...]


def _tc_node(h_n, r0, r1, v1n, v1r, c1, v2, d2, gam, bet, ws, wr):
    def body(hn, r0r, r1r, v1nr, v1rr, c1r, v2r, d2r, gr, br, wsr, wrr,
             hno, ab):
        yn = _node_core(hn[...], r0r, r1r, v1nr, v1rr, c1r, v2r, d2r, gr, br)
        hno[...] = yn
        ab[...] = jnp.concatenate(
            [_dot(yn, wsr[...]), _dot(yn, wrr[...])], axis=-1)

    c = lambda i: (0, 0)
    r = lambda i: (i, 0)
    return pl.pallas_call(
        body,
        grid=(N_NODES // NB,),
        in_specs=[
            pl.BlockSpec((NB, 64), r), pl.BlockSpec((NB, 64), r),
            pl.BlockSpec((NB, 64), r),
            pl.BlockSpec((64, 64), c), pl.BlockSpec((64, 64), c),
            pl.BlockSpec((1, 64), c),
            pl.BlockSpec((64, 64), c), pl.BlockSpec((1, 64), c),
            pl.BlockSpec((1, 64), c), pl.BlockSpec((1, 64), c),
            pl.BlockSpec((64, 64), c), pl.BlockSpec((64, 64), c),
        ],
        out_specs=[pl.BlockSpec((NB, 64), r), pl.BlockSpec((NB, 128), r)],
        out_shape=[jax.ShapeDtypeStruct((N_NODES, 64), jnp.float32),
                   jax.ShapeDtypeStruct((N_NODES, 128), jnp.float32)],
    )(h_n, r0, r1, v1n, v1r, c1, v2, d2, gam, bet, ws, wr)


def _tc_node_decode(h_n, r0, r1, v1n, v1r, c1, v2, d2, gam, bet,
                    dw1, db1, dw2, db2):
    def body(hn, r0r, r1r, v1nr, v1rr, c1r, v2r, d2r, gr, br,
             dw1r, db1r, dw2r, db2r, out):
        yn = _node_core(hn[...], r0r, r1r, v1nr, v1rr, c1r, v2r, d2r, gr, br)
        t = jax.nn.gelu(_dot(yn, dw1r[...]) + db1r[...])
        out[...] = _dot(t, dw2r[...]) + db2r[...]

    c = lambda i: (0, 0)
    r = lambda i: (i, 0)
    return pl.pallas_call(
        body,
        grid=(N_NODES // NB,),
        in_specs=[
            pl.BlockSpec((NB, 64), r), pl.BlockSpec((NB, 64), r),
            pl.BlockSpec((NB, 64), r),
            pl.BlockSpec((64, 64), c), pl.BlockSpec((64, 64), c),
            pl.BlockSpec((1, 64), c),
            pl.BlockSpec((64, 64), c), pl.BlockSpec((1, 64), c),
            pl.BlockSpec((1, 64), c), pl.BlockSpec((1, 64), c),
            pl.BlockSpec((64, 64), c), pl.BlockSpec((1, 64), c),
            pl.BlockSpec((64, 3), c), pl.BlockSpec((1, 3), c),
        ],
        out_specs=pl.BlockSpec((NB, 3), r),
        out_shape=jax.ShapeDtypeStruct((N_NODES, 3), jnp.float32),
    )(h_n, r0, r1, v1n, v1r, c1, v2, d2, gam, bet, dw1, db1, dw2, db2)


def _blockdiag(w):
    z = jnp.zeros_like(w)
    return jnp.concatenate(
        [jnp.concatenate([w, z], axis=1), jnp.concatenate([z, w], axis=1)],
        axis=0)


def _pair(b):
    return jnp.concatenate([b, b], axis=-1)


def kernel(nodes, edges, senders, receivers, globals_, params):
    p = params
    g = globals_.reshape(1, -1)
    row = lambda b: b.reshape(1, -1)

    en1, en2 = p["embed_node"]
    ee1, ee2 = p["embed_edge"]
    L = LATENT

    step_w = []
    for s in range(3):
        sp = p["steps"][s]
        W1, b1 = sp["edge"][0]["W"], sp["edge"][0]["b"]
        W2, b2 = sp["edge"][1]["W"], sp["edge"][1]["b"]
        V1, d1 = sp["node"][0]["W"], sp["node"][0]["b"]
        V2, d2 = sp["node"][1]["W"], sp["node"][1]["b"]
        step_w.append(dict(
            W1e=_blockdiag(W1[:L]), W1s=W1[L:2 * L], W1r=W1[2 * L:3 * L],
            c0=_pair(_dot(g, W1[3 * L:]) + b1),
            W2=_blockdiag(W2), b2=_pair(row(b2)),
            V1n=V1[:L], V1r=V1[L:2 * L],
            c1=_dot(g, V1[2 * L:]) + d1, V2=V2, d2=row(d2),
        ))

    gam, bet = row(p["ln_gamma"]), row(p["ln_beta"])
    dw1, db1 = p["decoder"][0]["W"], row(p["decoder"][0]["b"])
    dw2, db2 = p["decoder"][1]["W"], row(p["decoder"][1]["b"])

    snd2 = (senders * 2).reshape(NW, G_NCH, G_CH)
    rcv2 = (receivers * 2 + 1).reshape(NW, G_NCH, G_CH)
    rcv_s = receivers.reshape(NW, S_NCH, S_CH)
    zeros_n = jnp.zeros((N_ACC, LATENT), jnp.float32)

    h_n, ab = _tc_embed_node(
        nodes, en1["W"], row(en1["b"]), en2["W"], row(en2["b"]),
        step_w[0]["W1s"], step_w[0]["W1r"])
    edges_t = edges.T
    edges8t = edges_t.reshape(4, E2, 2).transpose(2, 0, 1).reshape(8, E2)
    h_e = _tc_embed_edge(
        edges8t, _blockdiag(ee1["W"]), _pair(row(ee1["b"])),
        _blockdiag(ee2["W"]), _pair(row(ee2["b"])))

    out = None
    for s in range(3):
        w = step_w[s]
        ga, gb = _sc_gather(ab.reshape(2 * N_NODES, L), snd2, rcv2)
        new_e = _tc_edge(h_e, ga.reshape(E2, 128), gb.reshape(E2, 128),
                         w["W1e"], w["c0"], w["W2"], w["b2"])
        r0, r1 = _sc_segsum(new_e.reshape(N_EDGES, L), rcv_s, zeros_n)
        if s < 2:
            nw = step_w[s + 1]
            h_n, ab = _tc_node(
                h_n, r0, r1, w["V1n"], w["V1r"], w["c1"], w["V2"], w["d2"],
                gam, bet, nw["W1s"], nw["W1r"])
        else:
            out = _tc_node_decode(
                h_n, r0, r1, w["V1n"], w["V1r"], w["c1"], w["V2"], w["d2"],
                gam, bet, dw1, db1, dw2, db2)
        h_e = new_e
    return out


# speedup vs baseline: 1.6546x; 1.6546x over previous
import functools

import jax
import jax.numpy as jnp
from jax import lax
from jax.experimental import pallas as pl
from jax.experimental.pallas import tpu as pltpu
from jax.experimental.pallas import tpu_sc as plsc

def _dot(a, b):
    return lax.dot_general(
        a.astype(jnp.bfloat16), b.astype(jnp.bfloat16),
        (((a.ndim - 1,), (0,)), ((), ())),
        preferred_element_type=jnp.float32)

N_NODES = 10000
N_EDGES = 320000
E2 = N_EDGES // 2
LATENT = 64

NC = 2
NS = 16
NW = NC * NS
E_PER_W = N_EDGES // NW
G_CH = 200
G_NCH = E_PER_W // G_CH
S_CH = 200
S_NCH = E_PER_W // S_CH
N_ACC = 10240
N_PER_S = N_ACC // NS

EB2 = 3200
NB = 2000

_SC_PARAMS = pltpu.CompilerParams(use_tc_tiling_on_sc=False)


def _mesh():
    return plsc.VectorSubcoreMesh(core_axis_name="c", subcore_axis_name="s",
                                  num_cores=NC, num_subcores=NS)


@functools.partial(
    pl.kernel,
    out_type=jax.ShapeDtypeStruct((N_EDGES, LATENT), jnp.float32),
    mesh=_mesh(),
    scratch_types=[
        pltpu.VMEM((G_NCH, G_CH), jnp.int32),
        pltpu.VMEM((G_NCH, G_CH), jnp.int32),
        pltpu.VMEM((G_CH, LATENT), jnp.float32),
        pltpu.VMEM((G_CH, LATENT), jnp.float32),
        pltpu.VMEM((G_CH, LATENT), jnp.float32),
        pltpu.VMEM((G_CH, LATENT), jnp.float32),
        pltpu.SemaphoreType.DMA,
        pltpu.SemaphoreType.DMA,
        pltpu.SemaphoreType.DMA,
        pltpu.SemaphoreType.DMA,
        pltpu.SemaphoreType.DMA,
        pltpu.SemaphoreType.DMA,
    ],
    compiler_params=_SC_PARAMS,
)
def _sc_gather(tab, snd_h, rcv_h, osum, ia, ib, ba0, bb0, ba1, bb1,
               sa0, sb0, sa1, sb1, wa0, wa1):
    wid = lax.axis_index("s") * NC + lax.axis_index("c")
    pltpu.sync_copy(snd_h.at[wid], ia)
    pltpu.sync_copy(rcv_h.at[wid], ib)
    base = wid * E_PER_W

    def _accum(dst, src):
        def addbody(r4, carry):
            for u in range(4):
                r = r4 * 4 + u
                for k in range(LATENT // 16):
                    sl = pl.ds(k * 16, 16)
                    dst[r, sl] = dst[r, sl] + src[r, sl]
            return carry
        lax.fori_loop(0, G_CH // 4, addbody, 0)

    def body(g, carry):
        i0 = 2 * g
        i1 = i0 + 1
        off0 = base + i0 * G_CH
        off1 = base + i1 * G_CH
        ca0 = pltpu.async_copy(tab.at[ia.at[i0]], ba0, sa0)
        cb0 = pltpu.async_copy(tab.at[ib.at[i0]], bb0, sb0)
        ca1 = pltpu.async_copy(tab.at[ia.at[i1]], ba1, sa1)
        cb1 = pltpu.async_copy(tab.at[ib.at[i1]], bb1, sb1)
        ca0.wait()
        cb0.wait()
        _accum(ba0, bb0)
        va0 = pltpu.async_copy(ba0, osum.at[pl.ds(off0, G_CH)], wa0)
        ca1.wait()
        cb1.wait()
        _accum(ba1, bb1)
        va1 = pltpu.async_copy(ba1, osum.at[pl.ds(off1, G_CH)], wa1)
        va0.wait()
        va1.wait()
        return carry

    lax.fori_loop(0, G_NCH // 2, body, 0)


@functools.partial(
    pl.kernel,
    out_type=(
        jax.ShapeDtypeStruct((N_ACC, LATENT), jnp.float32),
        jax.ShapeDtypeStruct((N_ACC, LATENT), jnp.float32),
    ),
    mesh=_mesh(),
    scratch_types=[
        pltpu.VMEM((S_NCH, S_CH), jnp.int32),
        pltpu.VMEM((2, S_CH, LATENT), jnp.float32),
        pltpu.VMEM_SHARED((N_ACC, LATENT), jnp.float32),
        pltpu.SemaphoreType.DMA,
        pltpu.SemaphoreType.DMA,
        pltpu.SemaphoreType.DMA,
        pltpu.SemaphoreType.DMA,
    ],
    compiler_params=_SC_PARAMS,
)
def _sc_segsum(vals_h, rcv_h, zeros_h, o0, o1, idx, buf, acc,
               l0, l1, t0, t1):
    cid = lax.axis_index("c")
    sid = lax.axis_index("s")
    wid = sid * NC + cid
    rows = pl.ds(sid * N_PER_S, N_PER_S)
    pltpu.sync_copy(zeros_h.at[rows], acc.at[rows])
    pltpu.sync_copy(rcv_h.at[wid], idx)
    plsc.subcore_barrier()
    base = wid * E_PER_W

    def body(g, carry):
        i0 = 2 * g
        i1 = i0 + 1
        cl0 = pltpu.async_copy(vals_h.at[pl.ds(base + i0 * S_CH, S_CH)],
                               buf.at[0], l0)
        cl1 = pltpu.async_copy(vals_h.at[pl.ds(base + i1 * S_CH, S_CH)],
                               buf.at[1], l1)
        cl0.wait()
        cs0 = pltpu.async_copy(buf.at[0], acc.at[idx.at[i0]], t0, add=True)
        cl1.wait()
        cs1 = pltpu.async_copy(buf.at[1], acc.at[idx.at[i1]], t1, add=True)
        cs0.wait()
        cs1.wait()
        return carry

    lax.fori_loop(0, S_NCH // 2, body, 0)
    plsc.subcore_barrier()

    @pl.when(cid == 0)
    def _():
        pltpu.sync_copy(acc.at[rows], o0.at[rows])

    @pl.when(cid == 1)
    def _():
        pltpu.sync_copy(acc.at[rows], o1.at[rows])


def _tc_embed_node(nodes, w1, b1, w2, b2, ws, wr):
    def body(x, w1r, b1r, w2r, b2r, wsr, wrr, hn, ab):
        h = jax.nn.gelu(_dot(x[...], w1r[...]) + b1r[...])
        hv = _dot(h, w2r[...]) + b2r[...]
        hn[...] = hv
        ab[...] = jnp.concatenate(
            [_dot(hv, wsr[...]), _dot(hv, wrr[...])], axis=-1)

    c = lambda i: (0, 0)
    r = lambda i: (i, 0)
    return pl.pallas_call(
        body,
        grid=(N_NODES // NB,),
        in_specs=[
            pl.BlockSpec((NB, 128), r),
            pl.BlockSpec((128, 64), c), pl.BlockSpec((1, 64), c),
            pl.BlockSpec((64, 64), c), pl.BlockSpec((1, 64), c),
            pl.BlockSpec((64, 64), c), pl.BlockSpec((64, 64), c),
        ],
        out_specs=[pl.BlockSpec((NB, 64), r), pl.BlockSpec((NB, 128), r)],
        out_shape=[jax.ShapeDtypeStruct((N_NODES, 64), jnp.float32),
                   jax.ShapeDtypeStruct((N_NODES, 128), jnp.float32)],
    )(nodes, w1, b1, w2, b2, ws, wr)


def _tc_embed_edge(edges8t, w1p, b1p, w2p, b2p):
    def body(x, w1r, b1r, w2r, b2r, he):
        pre = lax.dot_general(
            x[...].astype(jnp.bfloat16), w1r[...].astype(jnp.bfloat16),
            (((0,), (0,)), ((), ())), preferred_element_type=jnp.float32)
        h = jax.nn.gelu(pre + b1r[...])
        he[...] = _dot(h, w2r[...]) + b2r[...]

    c = lambda i: (0, 0)
    r = lambda i: (i, 0)
    return pl.pallas_call(
        body,
        grid=(E2 // EB2,),
        in_specs=[
            pl.BlockSpec((8, EB2), lambda i: (0, i)),
            pl.BlockSpec((8, 128), c), pl.BlockSpec((1, 128), c),
            pl.BlockSpec((128, 128), c), pl.BlockSpec((1, 128), c),
        ],
        out_specs=pl.BlockSpec((EB2, 128), r),
        out_shape=jax.ShapeDtypeStruct((E2, 128), jnp.float32),
    )(edges8t, w1p, b1p, w2p, b2p)


def _tc_edge(he_p, gs_p, w1p, c0p, w2p, b2p):
    def body(he, gsr, w1r, c0r, w2r, b2r, out):
        hev = he[...]
        pre = _dot(hev, w1r[...]) + gsr[...] + c0r[...]
        t = jax.nn.gelu(pre)
        out[...] = _dot(t, w2r[...]) + b2r[...] + hev

    c = lambda i: (0, 0)
    r = lambda i: (i, 0)
    return pl.pallas_call(
        body,
        grid=(E2 // EB2,),
        in_specs=[
            pl.BlockSpec((EB2, 128), r), pl.BlockSpec((EB2, 128), r),
            pl.BlockSpec((128, 128), c), pl.BlockSpec((1, 128), c),
            pl.BlockSpec((128, 128), c), pl.BlockSpec((1, 128), c),
        ],
        out_specs=pl.BlockSpec((EB2, 128), r),
        out_shape=jax.ShapeDtypeStruct((E2, 128), jnp.float32),
    )(he_p, gs_p, w1p, c0p, w2p, b2p)


def _node_core(hnv, r0, r1, v1n, v1r, c1, v2, d2, gam, bet):
    rec = r0[...] + r1[...]
    t = jax.nn.gelu(_dot(hnv, v1n[...]) + _dot(rec, v1r[...]) + c1[...])
    y = _dot(t, v2[...]) + d2[...] + hnv
    m = jnp.mean(y, axis=-1, keepdims=True)
    v = jnp.mean((y - m) ** 2, axis=-1, keepdims=True)
    return (y - m) / jnp.sqrt(v + 1e-6) * gam[...] + bet[...]


def _tc_node(h_n, r0, r1, v1n, v1r, c1, v2, d2, gam, bet, ws, wr):
    def body(hn, r0r, r1r, v1nr, v1rr, c1r, v2r, d2r, gr, br, wsr, wrr,
             hno, ab):
        yn = _node_core(hn[...], r0r, r1r, v1nr, v1rr, c1r, v2r, d2r, gr, br)
        hno[...] = yn
        ab[...] = jnp.concatenate(
            [_dot(yn, wsr[...]), _dot(yn, wrr[...])], axis=-1)

    c = lambda i: (0, 0)
    r = lambda i: (i, 0)
    return pl.pallas_call(
        body,
        grid=(N_NODES // NB,),
        in_specs=[
            pl.BlockSpec((NB, 64), r), pl.BlockSpec((NB, 64), r),
            pl.BlockSpec((NB, 64), r),
            pl.BlockSpec((64, 64), c), pl.BlockSpec((64, 64), c),
            pl.BlockSpec((1, 64), c),
            pl.BlockSpec((64, 64), c), pl.BlockSpec((1, 64), c),
            pl.BlockSpec((1, 64), c), pl.BlockSpec((1, 64), c),
            pl.BlockSpec((64, 64), c), pl.BlockSpec((64, 64), c),
        ],
        out_specs=[pl.BlockSpec((NB, 64), r), pl.BlockSpec((NB, 128), r)],
        out_shape=[jax.ShapeDtypeStruct((N_NODES, 64), jnp.float32),
                   jax.ShapeDtypeStruct((N_NODES, 128), jnp.float32)],
    )(h_n, r0, r1, v1n, v1r, c1, v2, d2, gam, bet, ws, wr)


def _tc_node_decode(h_n, r0, r1, v1n, v1r, c1, v2, d2, gam, bet,
                    dw1, db1, dw2, db2):
    def body(hn, r0r, r1r, v1nr, v1rr, c1r, v2r, d2r, gr, br,
             dw1r, db1r, dw2r, db2r, out):
        yn = _node_core(hn[...], r0r, r1r, v1nr, v1rr, c1r, v2r, d2r, gr, br)
        t = jax.nn.gelu(_dot(yn, dw1r[...]) + db1r[...])
        out[...] = _dot(t, dw2r[...]) + db2r[...]

    c = lambda i: (0, 0)
    r = lambda i: (i, 0)
    return pl.pallas_call(
        body,
        grid=(N_NODES // NB,),
        in_specs=[
            pl.BlockSpec((NB, 64), r), pl.BlockSpec((NB, 64), r),
            pl.BlockSpec((NB, 64), r),
            pl.BlockSpec((64, 64), c), pl.BlockSpec((64, 64), c),
            pl.BlockSpec((1, 64), c),
            pl.BlockSpec((64, 64), c), pl.BlockSpec((1, 64), c),
            pl.BlockSpec((1, 64), c), pl.BlockSpec((1, 64), c),
            pl.BlockSpec((64, 64), c), pl.BlockSpec((1, 64), c),
            pl.BlockSpec((64, 3), c), pl.BlockSpec((1, 3), c),
        ],
        out_specs=pl.BlockSpec((NB, 3), r),
        out_shape=jax.ShapeDtypeStruct((N_NODES, 3), jnp.float32),
    )(h_n, r0, r1, v1n, v1r, c1, v2, d2, gam, bet, dw1, db1, dw2, db2)


def _blockdiag(w):
    z = jnp.zeros_like(w)
    return jnp.concatenate(
        [jnp.concatenate([w, z], axis=1), jnp.concatenate([z, w], axis=1)],
        axis=0)


def _pair(b):
    return jnp.concatenate([b, b], axis=-1)


def kernel(nodes, edges, senders, receivers, globals_, params):
    p = params
    g = globals_.reshape(1, -1)
    row = lambda b: b.reshape(1, -1)

    en1, en2 = p["embed_node"]
    ee1, ee2 = p["embed_edge"]
    L = LATENT

    step_w = []
    for s in range(3):
        sp = p["steps"][s]
        W1, b1 = sp["edge"][0]["W"], sp["edge"][0]["b"]
        W2, b2 = sp["edge"][1]["W"], sp["edge"][1]["b"]
        V1, d1 = sp["node"][0]["W"], sp["node"][0]["b"]
        V2, d2 = sp["node"][1]["W"], sp["node"][1]["b"]
        step_w.append(dict(
            W1e=_blockdiag(W1[:L]), W1s=W1[L:2 * L], W1r=W1[2 * L:3 * L],
            c0=_pair(_dot(g, W1[3 * L:]) + b1),
            W2=_blockdiag(W2), b2=_pair(row(b2)),
            V1n=V1[:L], V1r=V1[L:2 * L],
            c1=_dot(g, V1[2 * L:]) + d1, V2=V2, d2=row(d2),
        ))

    gam, bet = row(p["ln_gamma"]), row(p["ln_beta"])
    dw1, db1 = p["decoder"][0]["W"], row(p["decoder"][0]["b"])
    dw2, db2 = p["decoder"][1]["W"], row(p["decoder"][1]["b"])

    snd2 = (senders * 2).reshape(NW, G_NCH, G_CH)
    rcv2 = (receivers * 2 + 1).reshape(NW, G_NCH, G_CH)
    rcv_s = receivers.reshape(NW, S_NCH, S_CH)
    zeros_n = jnp.zeros((N_ACC, LATENT), jnp.float32)

    h_n, ab = _tc_embed_node(
        nodes, en1["W"], row(en1["b"]), en2["W"], row(en2["b"]),
        step_w[0]["W1s"], step_w[0]["W1r"])
    edges_t = edges.T
    edges8t = edges_t.reshape(4, E2, 2).transpose(2, 0, 1).reshape(8, E2)
    h_e = _tc_embed_edge(
        edges8t, _blockdiag(ee1["W"]), _pair(row(ee1["b"])),
        _blockdiag(ee2["W"]), _pair(row(ee2["b"])))

    out = None
    for s in range(3):
        w = step_w[s]
        gs = _sc_gather(ab.reshape(2 * N_NODES, L), snd2, rcv2)
        new_e = _tc_edge(h_e, gs.reshape(E2, 128),
                         w["W1e"], w["c0"], w["W2"], w["b2"])
        r0, r1 = _sc_segsum(new_e.reshape(N_EDGES, L), rcv_s, zeros_n)
        if s < 2:
            nw = step_w[s + 1]
            h_n, ab = _tc_node(
                h_n, r0, r1, w["V1n"], w["V1r"], w["c1"], w["V2"], w["d2"],
                gam, bet, nw["W1s"], nw["W1r"])
        else:
            out = _tc_node_decode(
                h_n, r0, r1, w["V1n"], w["V1r"], w["c1"], w["V2"], w["d2"],
                gam, bet, dw1, db1, dw2, db2)
        h_e = new_e
    return out
